# baseline (speedup 1.0000x reference)
import jax
import jax.numpy as jnp
from jax.experimental import pallas as pl
from jax.experimental.pallas import tpu as pltpu
from functools import partial

HIDDEN = 768
NUM_EXPERTS = 64
TOP_K = 2


def _router_kernel(x_ref, w_ref, rw_ref, se_ref, bl_ref, uv_ref, gn_ref,
                   en_ref, usage_acc, ent_acc, *, n_tokens, n_steps):
    i = pl.program_id(0)

    @pl.when(i == 0)
    def _init():
        usage_acc[...] = jnp.zeros_like(usage_acc)
        ent_acc[...] = jnp.zeros_like(ent_acc)

    xblk = x_ref[...]
    w = w_ref[...]
    logits = jax.lax.dot_general(
        w, xblk, (((1,), (1,)), ((), ())),
        preferred_element_type=jnp.float32)

    e = logits.shape[0]
    ef = jnp.float32(e)
    iota = jax.lax.broadcasted_iota(jnp.int32, logits.shape, 0).astype(jnp.float32)

    m1 = jnp.max(logits, axis=0, keepdims=True)
    idx1 = jnp.min(jnp.where(logits == m1, iota, ef), axis=0, keepdims=True)
    masked = jnp.where(iota == idx1, -jnp.inf, logits)
    m2 = jnp.max(masked, axis=0, keepdims=True)
    idx2 = jnp.min(jnp.where(masked == m2, iota, ef), axis=0, keepdims=True)

    s = jnp.exp(m2 - m1)
    w1 = 1.0 / (1.0 + s)
    w2 = s / (1.0 + s)
    rw_ref[...] = jnp.concatenate([w1, w2], axis=0)
    se_ref[...] = jnp.concatenate([idx1, idx2], axis=0).astype(jnp.int32)

    t = logits - m1
    ex = jnp.exp(t)
    denom = jnp.sum(ex, axis=0, keepdims=True)
    inv = 1.0 / denom
    usage_acc[...] += ex * inv
    ent_acc[...] += jnp.log(denom) - jnp.sum(ex * t, axis=0, keepdims=True) * inv

    @pl.when(i == n_steps - 1)
    def _finalize():
        u = jnp.sum(usage_acc[...], axis=1, keepdims=True) / n_tokens
        su = jnp.sum(u)
        mean = su / e
        var = jnp.sum((u - mean) ** 2) / (e - 1)
        uv_ref[...] = var.reshape(1, 1)
        bl_ref[...] = (var * e).reshape(1, 1)
        en_ref[...] = (jnp.sum(ent_acc[...]) / n_tokens).reshape(1, 1)
        ut = u.reshape(1, e)
        lt = (ut < u).astype(jnp.float32)
        eq = (ut == u).astype(jnp.float32)
        c_lt = jnp.sum(lt, axis=1, keepdims=True)
        c_eq = jnp.sum(eq, axis=1, keepdims=True)
        ranksum = jnp.sum(u * (c_lt + (c_eq + 1.0) * 0.5))
        gn_ref[...] = (2.0 * ranksum / (e * su) - (e + 1.0) / e).reshape(1, 1)


@jax.jit
def kernel(x, W):
    b, s, h = x.shape
    n_tokens = b * s
    block = 2048
    n_steps = n_tokens // block
    xf = x.reshape(n_tokens, h)

    scalar_spec = pl.BlockSpec((1, 1), lambda i: (0, 0))
    out = pl.pallas_call(
        partial(_router_kernel, n_tokens=n_tokens, n_steps=n_steps),
        grid=(n_steps,),
        in_specs=[
            pl.BlockSpec((block, h), lambda i: (i, 0)),
            pl.BlockSpec((NUM_EXPERTS, h), lambda i: (0, 0)),
        ],
        out_specs=[
            pl.BlockSpec((TOP_K, block), lambda i: (0, i)),
            pl.BlockSpec((TOP_K, block), lambda i: (0, i)),
            scalar_spec, scalar_spec, scalar_spec, scalar_spec,
        ],
        out_shape=[
            jax.ShapeDtypeStruct((TOP_K, n_tokens), jnp.float32),
            jax.ShapeDtypeStruct((TOP_K, n_tokens), jnp.int32),
            jax.ShapeDtypeStruct((1, 1), jnp.float32),
            jax.ShapeDtypeStruct((1, 1), jnp.float32),
            jax.ShapeDtypeStruct((1, 1), jnp.float32),
            jax.ShapeDtypeStruct((1, 1), jnp.float32),
        ],
        scratch_shapes=[
            pltpu.VMEM((NUM_EXPERTS, block), jnp.float32),
            pltpu.VMEM((1, block), jnp.float32),
        ],
    )(xf, W)

    rw, se, bl, uv, gn, en = out
    return (rw.T.reshape(b, s, TOP_K), se.T.reshape(b, s, TOP_K),
            bl[0, 0], uv[0, 0], gn[0, 0], en[0, 0])


# speedup vs baseline: 1.0874x; 1.0874x over previous
import jax
import jax.numpy as jnp
from jax.experimental import pallas as pl
from jax.experimental.pallas import tpu as pltpu
from functools import partial

HIDDEN = 768
NUM_EXPERTS = 64
TOP_K = 2


def _router_kernel(x_ref, w_ref, rw_ref, se_ref, bl_ref, uv_ref, gn_ref,
                   en_ref, usage_acc, ent_acc, *, n_tokens, n_steps):
    i = pl.program_id(0)

    @pl.when(i == 0)
    def _init():
        usage_acc[...] = jnp.zeros_like(usage_acc)
        ent_acc[...] = jnp.zeros_like(ent_acc)

    xblk = x_ref[...]
    w = w_ref[...]
    logits = jax.lax.dot_general(
        w, xblk, (((1,), (1,)), ((), ())),
        preferred_element_type=jnp.float32)

    e = logits.shape[0]
    ef = jnp.float32(e)
    iota = jax.lax.broadcasted_iota(jnp.int32, logits.shape, 0).astype(jnp.float32)

    m1 = jnp.max(logits, axis=0, keepdims=True)
    idx1 = jnp.min(jnp.where(logits == m1, iota, ef), axis=0, keepdims=True)
    masked = jnp.where(iota == idx1, -jnp.inf, logits)
    m2 = jnp.max(masked, axis=0, keepdims=True)
    idx2 = jnp.min(jnp.where(masked == m2, iota, ef), axis=0, keepdims=True)

    s = jnp.exp(m2 - m1)
    w1 = 1.0 / (1.0 + s)
    w2 = s / (1.0 + s)
    rw_ref[...] = jnp.concatenate([w1, w2], axis=0)
    se_ref[...] = jnp.concatenate([idx1, idx2], axis=0).astype(jnp.int32)

    t = logits - m1
    ex = jnp.exp(t)
    denom = jnp.sum(ex, axis=0, keepdims=True)
    inv = 1.0 / denom
    usage_acc[...] += ex * inv
    ent_acc[...] += jnp.log(denom) - jnp.sum(ex * t, axis=0, keepdims=True) * inv

    @pl.when(i == n_steps - 1)
    def _finalize():
        u = jnp.sum(usage_acc[...], axis=1, keepdims=True) / n_tokens
        su = jnp.sum(u)
        mean = su / e
        var = jnp.sum((u - mean) ** 2) / (e - 1)
        uv_ref[...] = var.reshape(1, 1)
        bl_ref[...] = (var * e).reshape(1, 1)
        en_ref[...] = (jnp.sum(ent_acc[...]) / n_tokens).reshape(1, 1)
        ut = u.reshape(1, e)
        lt = (ut < u).astype(jnp.float32)
        eq = (ut == u).astype(jnp.float32)
        c_lt = jnp.sum(lt, axis=1, keepdims=True)
        c_eq = jnp.sum(eq, axis=1, keepdims=True)
        ranksum = jnp.sum(u * (c_lt + (c_eq + 1.0) * 0.5))
        gn_ref[...] = (2.0 * ranksum / (e * su) - (e + 1.0) / e).reshape(1, 1)


@jax.jit
def kernel(x, W):
    b, s, h = x.shape
    n_tokens = b * s
    block = 4096
    n_steps = n_tokens // block
    xf = x.reshape(n_tokens, h)

    scalar_spec = pl.BlockSpec((1, 1), lambda i: (0, 0))
    out = pl.pallas_call(
        partial(_router_kernel, n_tokens=n_tokens, n_steps=n_steps),
        grid=(n_steps,),
        in_specs=[
            pl.BlockSpec((block, h), lambda i: (i, 0)),
            pl.BlockSpec((NUM_EXPERTS, h), lambda i: (0, 0)),
        ],
        out_specs=[
            pl.BlockSpec((TOP_K, block), lambda i: (0, i)),
            pl.BlockSpec((TOP_K, block), lambda i: (0, i)),
            scalar_spec, scalar_spec, scalar_spec, scalar_spec,
        ],
        out_shape=[
            jax.ShapeDtypeStruct((TOP_K, n_tokens), jnp.float32),
            jax.ShapeDtypeStruct((TOP_K, n_tokens), jnp.int32),
            jax.ShapeDtypeStruct((1, 1), jnp.float32),
            jax.ShapeDtypeStruct((1, 1), jnp.float32),
            jax.ShapeDtypeStruct((1, 1), jnp.float32),
            jax.ShapeDtypeStruct((1, 1), jnp.float32),
        ],
        scratch_shapes=[
            pltpu.VMEM((NUM_EXPERTS, block), jnp.float32),
            pltpu.VMEM((1, block), jnp.float32),
        ],
    )(xf, W)

    rw, se, bl, uv, gn, en = out
    return (rw.T.reshape(b, s, TOP_K), se.T.reshape(b, s, TOP_K),
            bl[0, 0], uv[0, 0], gn[0, 0], en[0, 0])


# speedup vs baseline: 1.1335x; 1.0424x over previous
import jax
import jax.numpy as jnp
from jax.experimental import pallas as pl
from jax.experimental.pallas import tpu as pltpu
from functools import partial

HIDDEN = 768
NUM_EXPERTS = 64
TOP_K = 2


def _router_kernel(x_ref, w_ref, rw_ref, se_ref, bl_ref, uv_ref, gn_ref,
                   en_ref, usage_acc, ent_acc, *, n_tokens, n_steps):
    i = pl.program_id(0)

    @pl.when(i == 0)
    def _init():
        usage_acc[...] = jnp.zeros_like(usage_acc)
        ent_acc[...] = jnp.zeros_like(ent_acc)

    xblk = x_ref[...]
    w = w_ref[...]
    logits = jax.lax.dot_general(
        w, xblk, (((1,), (1,)), ((), ())),
        preferred_element_type=jnp.float32)

    e = logits.shape[0]
    ef = jnp.float32(e)
    iota = jax.lax.broadcasted_iota(jnp.int32, logits.shape, 0).astype(jnp.float32)

    m1 = jnp.max(logits, axis=0, keepdims=True)
    idx1 = jnp.min(jnp.where(logits == m1, iota, ef), axis=0, keepdims=True)
    masked = jnp.where(iota == idx1, -jnp.inf, logits)
    m2 = jnp.max(masked, axis=0, keepdims=True)
    idx2 = jnp.min(jnp.where(masked == m2, iota, ef), axis=0, keepdims=True)

    s = jnp.exp(m2 - m1)
    w1 = 1.0 / (1.0 + s)
    w2 = s / (1.0 + s)
    rw_ref[...] = jnp.concatenate([w1, w2], axis=0)
    se_ref[...] = jnp.concatenate([idx1, idx2], axis=0).astype(jnp.int32)

    t = logits - m1
    ex = jnp.exp(t)
    ones_e = jnp.ones((1, e), dtype=jnp.float32)
    denom = jax.lax.dot_general(
        ones_e, ex, (((1,), (0,)), ((), ())),
        preferred_element_type=jnp.float32)
    inv = 1.0 / denom
    usage_acc[...] += ex * inv
    ext_sum = jax.lax.dot_general(
        ones_e, ex * t, (((1,), (0,)), ((), ())),
        preferred_element_type=jnp.float32)
    ent_acc[...] += jnp.log(denom) - ext_sum * inv

    @pl.when(i == n_steps - 1)
    def _finalize():
        blk = usage_acc.shape[1]
        ones_b = jnp.ones((1, blk), dtype=jnp.float32)
        ut = jax.lax.dot_general(
            ones_b, usage_acc[...], (((1,), (1,)), ((), ())),
            preferred_element_type=jnp.float32) / n_tokens
        row_i = jax.lax.broadcasted_iota(jnp.int32, (e, e), 0)
        col_i = jax.lax.broadcasted_iota(jnp.int32, (e, e), 1)
        eye = (row_i == col_i).astype(jnp.float32)
        u = jax.lax.dot_general(
            eye, ut, (((1,), (1,)), ((), ())),
            preferred_element_type=jnp.float32)
        ent_tot = jax.lax.dot_general(
            ones_b, ent_acc[...], (((1,), (1,)), ((), ())),
            preferred_element_type=jnp.float32)
        su = jnp.sum(u)
        mean = su / e
        var = jnp.sum((u - mean) ** 2) / (e - 1)
        uv_ref[...] = var.reshape(1, 1)
        bl_ref[...] = (var * e).reshape(1, 1)
        en_ref[...] = ent_tot / n_tokens
        ones_col = jnp.ones((e, 1), dtype=jnp.float32)
        bm = jax.lax.dot_general(
            u, ones_e, (((1,), (0,)), ((), ())),
            preferred_element_type=jnp.float32)
        a = jnp.broadcast_to(ut, (e, e))
        lt = (a < bm).astype(jnp.float32)
        eq = (a == bm).astype(jnp.float32)
        c_lt = jax.lax.dot_general(
            lt, ones_col, (((1,), (0,)), ((), ())),
            preferred_element_type=jnp.float32)
        c_eq = jax.lax.dot_general(
            eq, ones_col, (((1,), (0,)), ((), ())),
            preferred_element_type=jnp.float32)
        ranksum = jnp.sum(u * (c_lt + (c_eq + 1.0) * 0.5))
        gn_ref[...] = (2.0 * ranksum / (e * su) - (e + 1.0) / e).reshape(1, 1)


@jax.jit
def kernel(x, W):
    b, s, h = x.shape
    n_tokens = b * s
    block = 4096
    n_steps = n_tokens // block
    xf = x.reshape(n_tokens, h)

    scalar_spec = pl.BlockSpec((1, 1), lambda i: (0, 0))
    out = pl.pallas_call(
        partial(_router_kernel, n_tokens=n_tokens, n_steps=n_steps),
        grid=(n_steps,),
        in_specs=[
            pl.BlockSpec((block, h), lambda i: (i, 0)),
            pl.BlockSpec((NUM_EXPERTS, h), lambda i: (0, 0)),
        ],
        out_specs=[
            pl.BlockSpec((TOP_K, block), lambda i: (0, i)),
            pl.BlockSpec((TOP_K, block), lambda i: (0, i)),
            scalar_spec, scalar_spec, scalar_spec, scalar_spec,
        ],
        out_shape=[
            jax.ShapeDtypeStruct((TOP_K, n_tokens), jnp.float32),
            jax.ShapeDtypeStruct((TOP_K, n_tokens), jnp.int32),
            jax.ShapeDtypeStruct((1, 1), jnp.float32),
            jax.ShapeDtypeStruct((1, 1), jnp.float32),
            jax.ShapeDtypeStruct((1, 1), jnp.float32),
            jax.ShapeDtypeStruct((1, 1), jnp.float32),
        ],
        scratch_shapes=[
            pltpu.VMEM((NUM_EXPERTS, block), jnp.float32),
            pltpu.VMEM((1, block), jnp.float32),
        ],
    )(xf, W)

    rw, se, bl, uv, gn, en = out
    return (rw.T.reshape(b, s, TOP_K), se.T.reshape(b, s, TOP_K),
            bl[0, 0], uv[0, 0], gn[0, 0], en[0, 0])
